# baseline (speedup 1.0000x reference)
import functools

import jax
import jax.numpy as jnp
from jax import lax
from jax.experimental import pallas as pl
from jax.experimental.pallas import tpu as pltpu
from jax.experimental.pallas import tpu_sc as plsc

_BATCH = 16384
_NW = 32
_BPW = _BATCH // _NW
_NU = 1000000
_NI = 100000
_CUT_U = (_NU // 128) * 128
_CUT_I = (_NI // 128) * 128
_WP_U = _CUT_U * 8
_WP_I = _CUT_I * 8

_SC_PARAMS = pltpu.CompilerParams(
    use_tc_tiling_on_sc=True, needs_layout_passes=False)
_MESH = plsc.VectorSubcoreMesh(core_axis_name="c", subcore_axis_name="s")


def _flat_view(tab, vcut):
    d = tab.shape[1]
    p = tab.T[:, :vcut]
    b = p.reshape(d // 8, 8, vcut // 128, 128)
    return b.transpose(0, 2, 1, 3).reshape(-1)


def _flat_view_half(tab, vcut, half):
    p = tab.T[16 * half:16 * half + 16, :vcut]
    b = p.reshape(2, 8, vcut // 128, 128)
    return b.transpose(0, 2, 1, 3).reshape(-1)


def _sc_gmf(uidx, iidx, fug, fig, ugt, igt):
    @functools.partial(
        pl.kernel,
        mesh=_MESH,
        compiler_params=_SC_PARAMS,
        out_type=[
            jax.ShapeDtypeStruct((8, _BATCH), jnp.float32),
            jax.ShapeDtypeStruct((8, _BATCH), jnp.float32),
        ],
        scratch_types=[
            pltpu.VMEM((_BPW + 16,), jnp.int32),
            pltpu.VMEM((_BPW + 16,), jnp.int32),
            pltpu.VMEM((8, 128), jnp.int32),
            pltpu.VMEM((8, 128), jnp.int32),
            pltpu.VMEM((8, 128), jnp.int32),
            pltpu.VMEM((8, 128), jnp.int32),
            pltpu.VMEM((8, 128), jnp.float32),
            pltpu.VMEM((8, _BPW), jnp.float32),
            pltpu.VMEM((8, _BPW), jnp.float32),
            pltpu.SemaphoreType.DMA,
            pltpu.SemaphoreType.DMA,
        ],
    )
    def k(uidx_hbm, iidx_hbm, fug_r, fig_r, ugt_r, igt_r, ug_o, ig_o,
          us, is_, ibu, ibi, ibu2, ibi2, tgb, ugv, igv, semA, semB):
        wid = lax.axis_index("s") * 2 + lax.axis_index("c")
        base = wid * _BPW
        pltpu.sync_copy(uidx_hbm.at[pl.ds(base, _BPW)], us.at[pl.ds(0, _BPW)])
        pltpu.sync_copy(iidx_hbm.at[pl.ds(base, _BPW)], is_.at[pl.ds(0, _BPW)])

        rows16 = lax.iota(jnp.int32, 16)
        m8 = rows16 < 8

        def build(cb, bu, bi):
            for t in range(8):
                uv = jnp.minimum(us[pl.ds(cb + t * 16, 16)], _CUT_U - 1)
                qv = jnp.minimum(is_[pl.ds(cb + t * 16, 16)], _CUT_I - 1)
                ju = (uv >> 7) * 1024 + (uv & 127)
                ji = (qv >> 7) * 1024 + (qv & 127)
                for c in range(8):
                    bu[c, pl.ds(t * 16, 16)] = ju + c * 128
                    bi[c, pl.ds(t * 16, 16)] = ji + c * 128

        def copies(cb, bu, bi, sem):
            out = []
            for c in range(8):
                out.append(pltpu.make_async_copy(
                    fug_r.at[bu.at[c]], ugv.at[c, pl.ds(cb, 128)], sem))
                out.append(pltpu.make_async_copy(
                    fig_r.at[bi.at[c]], igv.at[c, pl.ds(cb, 128)], sem))
            return out

        def superchunk(g, carry):
            cprev = (g - 1) * 256
            c0 = g * 256

            @pl.when(g > 0)
            def _():
                for cp in copies(cprev, ibu, ibi, semA):
                    cp.wait()

            build(c0, ibu, ibi)
            for cp in copies(c0, ibu, ibi, semA):
                cp.start()

            @pl.when(g > 0)
            def _():
                for cp in copies(cprev + 128, ibu2, ibi2, semB):
                    cp.wait()

            build(c0 + 128, ibu2, ibi2)
            for cp in copies(c0 + 128, ibu2, ibi2, semB):
                cp.start()
            return carry

        nsuper = _BPW // 256
        lax.fori_loop(0, nsuper, superchunk, 0)
        clast = (nsuper - 1) * 256
        for cp in copies(clast, ibu, ibi, semA):
            cp.wait()
        for cp in copies(clast + 128, ibu2, ibi2, semB):
            cp.wait()

        def fixup(gg, carry):
            gb = gg * 16
            uv = us[pl.ds(gb, 16)]
            qv = is_[pl.ds(gb, 16)]

            @pl.when(jnp.max(uv, axis=0) >= _CUT_U)
            def _():
                for e in range(16):
                    r = uv[e]

                    @pl.when(r >= _CUT_U)
                    def _():
                        ju = pl.multiple_of((r >> 7) << 7, 128)
                        pltpu.sync_copy(ugt_r.at[:, pl.ds(ju, 128)], tgb)
                        k16 = jnp.full((16,), gb + e, jnp.int32)
                        l16 = jnp.full((16,), r & 127, jnp.int32)
                        vg = plsc.load_gather(tgb, [rows16, l16], mask=m8)
                        plsc.store_scatter(ugv, [rows16, k16], vg, mask=m8)

            @pl.when(jnp.max(qv, axis=0) >= _CUT_I)
            def _():
                for e in range(16):
                    q = qv[e]

                    @pl.when(q >= _CUT_I)
                    def _():
                        ji = pl.multiple_of((q >> 7) << 7, 128)
                        pltpu.sync_copy(igt_r.at[:, pl.ds(ji, 128)], tgb)
                        k16 = jnp.full((16,), gb + e, jnp.int32)
                        l16 = jnp.full((16,), q & 127, jnp.int32)
                        vg = plsc.load_gather(tgb, [rows16, l16], mask=m8)
                        plsc.store_scatter(igv, [rows16, k16], vg, mask=m8)
            return carry

        lax.fori_loop(0, _BPW // 16, fixup, 0)

        pltpu.sync_copy(ugv, ug_o.at[:, pl.ds(base, _BPW)])
        pltpu.sync_copy(igv, ig_o.at[:, pl.ds(base, _BPW)])

    return k(uidx, iidx, fug, fig, ugt, igt)


def _sc_mlp(uidx, iidx, fum, fim, umt, imt, half):
    @functools.partial(
        pl.kernel,
        mesh=_MESH,
        compiler_params=_SC_PARAMS,
        out_type=[
            jax.ShapeDtypeStruct((16, _BATCH), jnp.float32),
            jax.ShapeDtypeStruct((16, _BATCH), jnp.float32),
        ],
        scratch_types=[
            pltpu.VMEM((_BPW + 16,), jnp.int32),
            pltpu.VMEM((_BPW + 16,), jnp.int32),
            pltpu.VMEM((16, 128), jnp.int32),
            pltpu.VMEM((16, 128), jnp.int32),
            pltpu.VMEM((16, 128), jnp.int32),
            pltpu.VMEM((16, 128), jnp.int32),
            pltpu.VMEM((32, 128), jnp.float32),
            pltpu.VMEM((16, _BPW), jnp.float32),
            pltpu.VMEM((16, _BPW), jnp.float32),
            pltpu.SemaphoreType.DMA,
            pltpu.SemaphoreType.DMA,
        ],
    )
    def k(uidx_hbm, iidx_hbm, fum_r, fim_r, umt_r, imt_r,
          um_o, im_o,
          us, is_, ibu, ibi, ibu2, ibi2, tmb, umv, imv, semA, semB):
        wid = lax.axis_index("s") * 2 + lax.axis_index("c")
        base = wid * _BPW
        pltpu.sync_copy(uidx_hbm.at[pl.ds(base, _BPW)], us.at[pl.ds(0, _BPW)])
        pltpu.sync_copy(iidx_hbm.at[pl.ds(base, _BPW)], is_.at[pl.ds(0, _BPW)])

        rows16 = lax.iota(jnp.int32, 16)

        def build(cb, bu, bi):
            for t in range(8):
                uv = jnp.minimum(us[pl.ds(cb + t * 16, 16)], _CUT_U - 1)
                qv = jnp.minimum(is_[pl.ds(cb + t * 16, 16)], _CUT_I - 1)
                ju = (uv >> 7) * 1024 + (uv & 127)
                ji = (qv >> 7) * 1024 + (qv & 127)
                for c in range(16):
                    bu[c, pl.ds(t * 16, 16)] = (
                        ju + (c // 8) * _WP_U + (c % 8) * 128)
                    bi[c, pl.ds(t * 16, 16)] = (
                        ji + (c // 8) * _WP_I + (c % 8) * 128)

        def copies(cb, bu, bi, sem):
            out = []
            for c in range(16):
                out.append(pltpu.make_async_copy(
                    fum_r.at[bu.at[c]], umv.at[c, pl.ds(cb, 128)], sem))
                out.append(pltpu.make_async_copy(
                    fim_r.at[bi.at[c]], imv.at[c, pl.ds(cb, 128)], sem))
            return out

        def superchunk(g, carry):
            cprev = (g - 1) * 256
            c0 = g * 256

            @pl.when(g > 0)
            def _():
                for cp in copies(cprev, ibu, ibi, semA):
                    cp.wait()

            build(c0, ibu, ibi)
            for cp in copies(c0, ibu, ibi, semA):
                cp.start()

            @pl.when(g > 0)
            def _():
                for cp in copies(cprev + 128, ibu2, ibi2, semB):
                    cp.wait()

            build(c0 + 128, ibu2, ibi2)
            for cp in copies(c0 + 128, ibu2, ibi2, semB):
                cp.start()
            return carry

        nsuper = _BPW // 256
        lax.fori_loop(0, nsuper, superchunk, 0)
        clast = (nsuper - 1) * 256
        for cp in copies(clast, ibu, ibi, semA):
            cp.wait()
        for cp in copies(clast + 128, ibu2, ibi2, semB):
            cp.wait()

        def fixup(gg, carry):
            gb = gg * 16
            uv = us[pl.ds(gb, 16)]
            qv = is_[pl.ds(gb, 16)]

            @pl.when(jnp.max(uv, axis=0) >= _CUT_U)
            def _():
                for e in range(16):
                    r = uv[e]

                    @pl.when(r >= _CUT_U)
                    def _():
                        ju = pl.multiple_of((r >> 7) << 7, 128)
                        pltpu.sync_copy(umt_r.at[:, pl.ds(ju, 128)], tmb)
                        k16 = jnp.full((16,), gb + e, jnp.int32)
                        l16 = jnp.full((16,), r & 127, jnp.int32)
                        vm = plsc.load_gather(
                            tmb, [rows16 + 16 * half, l16])
                        plsc.store_scatter(umv, [rows16, k16], vm)

            @pl.when(jnp.max(qv, axis=0) >= _CUT_I)
            def _():
                for e in range(16):
                    q = qv[e]

                    @pl.when(q >= _CUT_I)
                    def _():
                        ji = pl.multiple_of((q >> 7) << 7, 128)
                        pltpu.sync_copy(imt_r.at[:, pl.ds(ji, 128)], tmb)
                        k16 = jnp.full((16,), gb + e, jnp.int32)
                        l16 = jnp.full((16,), q & 127, jnp.int32)
                        vm = plsc.load_gather(
                            tmb, [rows16 + 16 * half, l16])
                        plsc.store_scatter(imv, [rows16, k16], vm)
            return carry

        lax.fori_loop(0, _BPW // 16, fixup, 0)

        pltpu.sync_copy(umv, um_o.at[:, pl.ds(base, _BPW)])
        pltpu.sync_copy(imv, im_o.at[:, pl.ds(base, _BPW)])

    return k(uidx, iidx, fum, fim, umt, imt)


def _tc_head_t(ugT, igT, umTa, umTb, imTa, imTb, W1uTa, W1uTb, W1iTa, W1iTb,
               b1c, W2T, b2c, W3T, b3c, Wfg, Wfh, bf):
    bb = 2048
    grid = (_BATCH // bb,)

    def body(ug_r, ig_r, uma_r, umb_r, ima_r, imb_r,
             w1ua_r, w1ub_r, w1ia_r, w1ib_r, b1_r, w2_r, b2_r,
             w3_r, b3_r, wfg_r, wfh_r, bf_r, o_r):
        g = ug_r[...] * ig_r[...]
        h = jnp.dot(w1ua_r[...], uma_r[...], preferred_element_type=jnp.float32)
        h = h + jnp.dot(w1ub_r[...], umb_r[...], preferred_element_type=jnp.float32)
        h = h + jnp.dot(w1ia_r[...], ima_r[...], preferred_element_type=jnp.float32)
        h = h + jnp.dot(w1ib_r[...], imb_r[...], preferred_element_type=jnp.float32)
        h = jnp.maximum(h + b1_r[...], 0.0)
        h = jnp.maximum(
            jnp.dot(w2_r[...], h, preferred_element_type=jnp.float32)
            + b2_r[...], 0.0)
        h = jnp.maximum(
            jnp.dot(w3_r[...], h, preferred_element_type=jnp.float32)
            + b3_r[...], 0.0)
        dn = (((0,), (0,)), ((), ()))
        s = lax.dot_general(wfg_r[...], g, dn,
                            preferred_element_type=jnp.float32)
        s = s + lax.dot_general(wfh_r[...], h, dn,
                                preferred_element_type=jnp.float32)
        s = s + bf_r[...]
        o_r[...] = jax.nn.sigmoid(s)[0]

    batch_spec = lambda d: pl.BlockSpec((d, bb), lambda i: (0, i))
    full_spec = lambda a: pl.BlockSpec(a.shape, lambda i: (0,) * a.ndim)
    return pl.pallas_call(
        body,
        grid=grid,
        in_specs=[
            batch_spec(8), batch_spec(8), batch_spec(16), batch_spec(16),
            batch_spec(16), batch_spec(16),
            full_spec(W1uTa), full_spec(W1uTb),
            full_spec(W1iTa), full_spec(W1iTb), full_spec(b1c),
            full_spec(W2T), full_spec(b2c), full_spec(W3T), full_spec(b3c),
            full_spec(Wfg), full_spec(Wfh), full_spec(bf),
        ],
        out_specs=pl.BlockSpec((bb,), lambda i: (i,)),
        out_shape=jax.ShapeDtypeStruct((_BATCH,), jnp.float32),
    )(ugT, igT, umTa, umTb, imTa, imTb, W1uTa, W1uTb, W1iTa, W1iTb,
      b1c, W2T, b2c, W3T, b3c, Wfg, Wfh, bf)


def kernel(user_indices, item_indices, U_gmf, I_gmf, U_mlp, I_mlp,
           W1, b1, W2, b2, W3, b3, Wf, bf):
    uidx = user_indices.astype(jnp.int32)
    iidx = item_indices.astype(jnp.int32)
    ugT, igT = _sc_gmf(
        uidx, iidx,
        _flat_view(U_gmf, _CUT_U), _flat_view(I_gmf, _CUT_I),
        U_gmf.T, I_gmf.T)
    umTa, imTa = _sc_mlp(
        uidx, iidx,
        _flat_view_half(U_mlp, _CUT_U, 0), _flat_view_half(I_mlp, _CUT_I, 0),
        U_mlp.T, I_mlp.T, 0)
    umTb, imTb = _sc_mlp(
        uidx, iidx,
        _flat_view_half(U_mlp, _CUT_U, 1), _flat_view_half(I_mlp, _CUT_I, 1),
        U_mlp.T, I_mlp.T, 1)
    w1u = W1[:32].T
    w1i = W1[32:].T
    out = _tc_head_t(
        ugT, igT, umTa, umTb, imTa, imTb,
        w1u[:, :16], w1u[:, 16:], w1i[:, :16], w1i[:, 16:],
        b1.reshape(32, 1),
        W2.T, b2.reshape(16, 1), W3.T, b3.reshape(8, 1),
        Wf[:8], Wf[8:], bf.reshape(1, 1),
    )
    return out.reshape(_BATCH, 1)


# speedup vs baseline: 1.0724x; 1.0724x over previous
import functools

import jax
import jax.numpy as jnp
from jax import lax
from jax.experimental import pallas as pl
from jax.experimental.pallas import tpu as pltpu
from jax.experimental.pallas import tpu_sc as plsc

_BATCH = 16384
_NW = 32
_BPW = _BATCH // _NW
_NU = 1000000
_NI = 100000
_CUT_U = (_NU // 128) * 128
_CUT_I = (_NI // 128) * 128
_WP_U = _CUT_U * 8
_WP_I = _CUT_I * 8

_SC_PARAMS = pltpu.CompilerParams(
    use_tc_tiling_on_sc=True, needs_layout_passes=False)
_MESH = plsc.VectorSubcoreMesh(core_axis_name="c", subcore_axis_name="s")


def _flat_view(tab, vcut):
    d = tab.shape[1]
    p = tab.T[:, :vcut]
    b = p.reshape(d // 8, 8, vcut // 128, 128)
    return b.transpose(0, 2, 1, 3).reshape(-1)


def _sc_gmf(uidx, iidx, fug, fig, ugt, igt):
    @functools.partial(
        pl.kernel,
        mesh=_MESH,
        compiler_params=_SC_PARAMS,
        out_type=[
            jax.ShapeDtypeStruct((8, _BATCH), jnp.float32),
            jax.ShapeDtypeStruct((8, _BATCH), jnp.float32),
        ],
        scratch_types=[
            pltpu.VMEM((_BPW + 16,), jnp.int32),
            pltpu.VMEM((_BPW + 16,), jnp.int32),
            pltpu.VMEM((8, 128), jnp.int32),
            pltpu.VMEM((8, 128), jnp.int32),
            pltpu.VMEM((8, 128), jnp.int32),
            pltpu.VMEM((8, 128), jnp.int32),
            pltpu.VMEM((8, 128), jnp.float32),
            pltpu.VMEM((8, _BPW), jnp.float32),
            pltpu.VMEM((8, _BPW), jnp.float32),
            pltpu.SemaphoreType.DMA,
            pltpu.SemaphoreType.DMA,
        ],
    )
    def k(uidx_hbm, iidx_hbm, fug_r, fig_r, ugt_r, igt_r, ug_o, ig_o,
          us, is_, ibu, ibi, ibu2, ibi2, tgb, ugv, igv, semA, semB):
        wid = lax.axis_index("s") * 2 + lax.axis_index("c")
        base = wid * _BPW
        pltpu.sync_copy(uidx_hbm.at[pl.ds(base, _BPW)], us.at[pl.ds(0, _BPW)])
        pltpu.sync_copy(iidx_hbm.at[pl.ds(base, _BPW)], is_.at[pl.ds(0, _BPW)])

        rows16 = lax.iota(jnp.int32, 16)
        m8 = rows16 < 8

        def build(cb, bu, bi):
            for t in range(8):
                uv = jnp.minimum(us[pl.ds(cb + t * 16, 16)], _CUT_U - 1)
                qv = jnp.minimum(is_[pl.ds(cb + t * 16, 16)], _CUT_I - 1)
                ju = (uv >> 7) * 1024 + (uv & 127)
                ji = (qv >> 7) * 1024 + (qv & 127)
                for c in range(8):
                    bu[c, pl.ds(t * 16, 16)] = ju + c * 128
                    bi[c, pl.ds(t * 16, 16)] = ji + c * 128

        def copies(cb, bu, bi, sem):
            out = []
            for c in range(8):
                out.append(pltpu.make_async_copy(
                    fug_r.at[bu.at[c]], ugv.at[c, pl.ds(cb, 128)], sem))
                out.append(pltpu.make_async_copy(
                    fig_r.at[bi.at[c]], igv.at[c, pl.ds(cb, 128)], sem))
            return out

        def superchunk(g, carry):
            cprev = (g - 1) * 256
            c0 = g * 256

            @pl.when(g > 0)
            def _():
                for cp in copies(cprev, ibu, ibi, semA):
                    cp.wait()

            build(c0, ibu, ibi)
            for cp in copies(c0, ibu, ibi, semA):
                cp.start()

            @pl.when(g > 0)
            def _():
                for cp in copies(cprev + 128, ibu2, ibi2, semB):
                    cp.wait()

            build(c0 + 128, ibu2, ibi2)
            for cp in copies(c0 + 128, ibu2, ibi2, semB):
                cp.start()
            return carry

        nsuper = _BPW // 256
        lax.fori_loop(0, nsuper, superchunk, 0)
        clast = (nsuper - 1) * 256
        for cp in copies(clast, ibu, ibi, semA):
            cp.wait()
        for cp in copies(clast + 128, ibu2, ibi2, semB):
            cp.wait()

        def fixup(gg, carry):
            gb = gg * 16
            uv = us[pl.ds(gb, 16)]
            qv = is_[pl.ds(gb, 16)]

            @pl.when(jnp.max(uv, axis=0) >= _CUT_U)
            def _():
                for e in range(16):
                    r = uv[e]

                    @pl.when(r >= _CUT_U)
                    def _():
                        ju = pl.multiple_of((r >> 7) << 7, 128)
                        pltpu.sync_copy(ugt_r.at[:, pl.ds(ju, 128)], tgb)
                        k16 = jnp.full((16,), gb + e, jnp.int32)
                        l16 = jnp.full((16,), r & 127, jnp.int32)
                        vg = plsc.load_gather(tgb, [rows16, l16], mask=m8)
                        plsc.store_scatter(ugv, [rows16, k16], vg, mask=m8)

            @pl.when(jnp.max(qv, axis=0) >= _CUT_I)
            def _():
                for e in range(16):
                    q = qv[e]

                    @pl.when(q >= _CUT_I)
                    def _():
                        ji = pl.multiple_of((q >> 7) << 7, 128)
                        pltpu.sync_copy(igt_r.at[:, pl.ds(ji, 128)], tgb)
                        k16 = jnp.full((16,), gb + e, jnp.int32)
                        l16 = jnp.full((16,), q & 127, jnp.int32)
                        vg = plsc.load_gather(tgb, [rows16, l16], mask=m8)
                        plsc.store_scatter(igv, [rows16, k16], vg, mask=m8)
            return carry

        lax.fori_loop(0, _BPW // 16, fixup, 0)

        pltpu.sync_copy(ugv, ug_o.at[:, pl.ds(base, _BPW)])
        pltpu.sync_copy(igv, ig_o.at[:, pl.ds(base, _BPW)])

    return k(uidx, iidx, fug, fig, ugt, igt)


def _sc_mlp(uidx, iidx, fum, fim, umt, imt):
    @functools.partial(
        pl.kernel,
        mesh=_MESH,
        compiler_params=_SC_PARAMS,
        out_type=[
            jax.ShapeDtypeStruct((32, _BATCH), jnp.float32),
            jax.ShapeDtypeStruct((32, _BATCH), jnp.float32),
        ],
        scratch_types=[
            pltpu.VMEM((_BPW + 16,), jnp.int32),
            pltpu.VMEM((_BPW + 16,), jnp.int32),
            pltpu.VMEM((32, 128), jnp.int32),
            pltpu.VMEM((32, 128), jnp.int32),
            pltpu.VMEM((32, 128), jnp.int32),
            pltpu.VMEM((32, 128), jnp.int32),
            pltpu.VMEM((32, 128), jnp.float32),
            pltpu.VMEM((32, _BPW), jnp.float32),
            pltpu.VMEM((32, _BPW), jnp.float32),
            pltpu.SemaphoreType.DMA,
            pltpu.SemaphoreType.DMA,
        ],
    )
    def k(uidx_hbm, iidx_hbm, fum_r, fim_r, umt_r, imt_r,
          um_o, im_o,
          us, is_, ibu, ibi, ibu2, ibi2, tmb, umv, imv, semA, semB):
        wid = lax.axis_index("s") * 2 + lax.axis_index("c")
        base = wid * _BPW
        pltpu.sync_copy(uidx_hbm.at[pl.ds(base, _BPW)], us.at[pl.ds(0, _BPW)])
        pltpu.sync_copy(iidx_hbm.at[pl.ds(base, _BPW)], is_.at[pl.ds(0, _BPW)])

        rows16 = lax.iota(jnp.int32, 16)

        def build(cb, bu, bi):
            for t in range(8):
                uv = jnp.minimum(us[pl.ds(cb + t * 16, 16)], _CUT_U - 1)
                qv = jnp.minimum(is_[pl.ds(cb + t * 16, 16)], _CUT_I - 1)
                ju = (uv >> 7) * 1024 + (uv & 127)
                ji = (qv >> 7) * 1024 + (qv & 127)
                for c in range(32):
                    bu[c, pl.ds(t * 16, 16)] = (
                        ju + (c // 8) * _WP_U + (c % 8) * 128)
                    bi[c, pl.ds(t * 16, 16)] = (
                        ji + (c // 8) * _WP_I + (c % 8) * 128)

        def copies(cb, bu, bi, sem):
            out = []
            for c in range(32):
                out.append(pltpu.make_async_copy(
                    fum_r.at[bu.at[c]], umv.at[c, pl.ds(cb, 128)], sem))
                out.append(pltpu.make_async_copy(
                    fim_r.at[bi.at[c]], imv.at[c, pl.ds(cb, 128)], sem))
            return out

        def superchunk(g, carry):
            cprev = (g - 1) * 256
            c0 = g * 256

            @pl.when(g > 0)
            def _():
                for cp in copies(cprev, ibu, ibi, semA):
                    cp.wait()

            build(c0, ibu, ibi)
            for cp in copies(c0, ibu, ibi, semA):
                cp.start()

            @pl.when(g > 0)
            def _():
                for cp in copies(cprev + 128, ibu2, ibi2, semB):
                    cp.wait()

            build(c0 + 128, ibu2, ibi2)
            for cp in copies(c0 + 128, ibu2, ibi2, semB):
                cp.start()
            return carry

        nsuper = _BPW // 256
        lax.fori_loop(0, nsuper, superchunk, 0)
        clast = (nsuper - 1) * 256
        for cp in copies(clast, ibu, ibi, semA):
            cp.wait()
        for cp in copies(clast + 128, ibu2, ibi2, semB):
            cp.wait()

        def fixup(gg, carry):
            gb = gg * 16
            uv = us[pl.ds(gb, 16)]
            qv = is_[pl.ds(gb, 16)]

            @pl.when(jnp.max(uv, axis=0) >= _CUT_U)
            def _():
                for e in range(16):
                    r = uv[e]

                    @pl.when(r >= _CUT_U)
                    def _():
                        ju = pl.multiple_of((r >> 7) << 7, 128)
                        pltpu.sync_copy(umt_r.at[:, pl.ds(ju, 128)], tmb)
                        k16 = jnp.full((16,), gb + e, jnp.int32)
                        l16 = jnp.full((16,), r & 127, jnp.int32)
                        for h in range(2):
                            rh = rows16 + (16 * h)
                            vm = plsc.load_gather(tmb, [rh, l16])
                            plsc.store_scatter(umv, [rh, k16], vm)

            @pl.when(jnp.max(qv, axis=0) >= _CUT_I)
            def _():
                for e in range(16):
                    q = qv[e]

                    @pl.when(q >= _CUT_I)
                    def _():
                        ji = pl.multiple_of((q >> 7) << 7, 128)
                        pltpu.sync_copy(imt_r.at[:, pl.ds(ji, 128)], tmb)
                        k16 = jnp.full((16,), gb + e, jnp.int32)
                        l16 = jnp.full((16,), q & 127, jnp.int32)
                        for h in range(2):
                            rh = rows16 + (16 * h)
                            vm = plsc.load_gather(tmb, [rh, l16])
                            plsc.store_scatter(imv, [rh, k16], vm)
            return carry

        lax.fori_loop(0, _BPW // 16, fixup, 0)

        pltpu.sync_copy(umv, um_o.at[:, pl.ds(base, _BPW)])
        pltpu.sync_copy(imv, im_o.at[:, pl.ds(base, _BPW)])

    return k(uidx, iidx, fum, fim, umt, imt)


def _tc_head_t(ugT, igT, umT, imT, W1uT, W1iT, b1c, W2T, b2c, W3T, b3c,
               Wfg, Wfh, bf):
    bb = 2048
    grid = (_BATCH // bb,)

    def body(ug_r, ig_r, um_r, im_r, w1u_r, w1i_r, b1_r, w2_r, b2_r,
             w3_r, b3_r, wfg_r, wfh_r, bf_r, o_r):
        g = ug_r[...] * ig_r[...]
        h = jnp.dot(w1u_r[...], um_r[...], preferred_element_type=jnp.float32)
        h = h + jnp.dot(w1i_r[...], im_r[...], preferred_element_type=jnp.float32)
        h = jnp.maximum(h + b1_r[...], 0.0)
        h = jnp.maximum(
            jnp.dot(w2_r[...], h, preferred_element_type=jnp.float32)
            + b2_r[...], 0.0)
        h = jnp.maximum(
            jnp.dot(w3_r[...], h, preferred_element_type=jnp.float32)
            + b3_r[...], 0.0)
        dn = (((0,), (0,)), ((), ()))
        s = lax.dot_general(wfg_r[...], g, dn,
                            preferred_element_type=jnp.float32)
        s = s + lax.dot_general(wfh_r[...], h, dn,
                                preferred_element_type=jnp.float32)
        s = s + bf_r[...]
        o_r[...] = jax.nn.sigmoid(s)[0]

    batch_spec = lambda d: pl.BlockSpec((d, bb), lambda i: (0, i))
    full_spec = lambda a: pl.BlockSpec(a.shape, lambda i: (0,) * a.ndim)
    return pl.pallas_call(
        body,
        grid=grid,
        in_specs=[
            batch_spec(8), batch_spec(8), batch_spec(32), batch_spec(32),
            full_spec(W1uT), full_spec(W1iT), full_spec(b1c),
            full_spec(W2T), full_spec(b2c), full_spec(W3T), full_spec(b3c),
            full_spec(Wfg), full_spec(Wfh), full_spec(bf),
        ],
        out_specs=pl.BlockSpec((bb,), lambda i: (i,)),
        out_shape=jax.ShapeDtypeStruct((_BATCH,), jnp.float32),
    )(ugT, igT, umT, imT, W1uT, W1iT, b1c, W2T, b2c, W3T, b3c, Wfg, Wfh, bf)


def kernel(user_indices, item_indices, U_gmf, I_gmf, U_mlp, I_mlp,
           W1, b1, W2, b2, W3, b3, Wf, bf):
    uidx = user_indices.astype(jnp.int32)
    iidx = item_indices.astype(jnp.int32)
    ugT, igT = _sc_gmf(
        uidx, iidx,
        _flat_view(U_gmf, _CUT_U), _flat_view(I_gmf, _CUT_I),
        U_gmf.T, I_gmf.T)
    umT, imT = _sc_mlp(
        uidx, iidx,
        _flat_view(U_mlp, _CUT_U), _flat_view(I_mlp, _CUT_I),
        U_mlp.T, I_mlp.T)
    out = _tc_head_t(
        ugT, igT, umT, imT,
        W1[:32].T, W1[32:].T, b1.reshape(32, 1),
        W2.T, b2.reshape(16, 1), W3.T, b3.reshape(8, 1),
        Wf[:8], Wf[8:], bf.reshape(1, 1),
    )
    return out.reshape(_BATCH, 1)
